# Initial kernel scaffold; baseline (speedup 1.0000x reference)
import functools

import jax
import jax.numpy as jnp
from jax import lax
from jax.experimental import pallas as pl
from jax.experimental.pallas import tpu as pltpu
from jax.experimental.pallas import tpu_sc as plsc

N = 10000
E = 320000
D = 128

NC = 2
NS = 16
NW = NC * NS
EPW = E // NW
K = 125
CH = EPW // K
RPW = N // NS


def _seg_sum_body(with_deg, feat, src3, dst3, out_p, *rest):
    if with_deg:
        deg_out, acc, srcb, dstb, rows, dega, degz, ones = rest
    else:
        acc, srcb, dstb, rows = rest
    c = lax.axis_index("c")
    s = lax.axis_index("s")
    w = c * NS + s

    @pl.loop(0, K)
    def _(j):
        @pl.loop(0, D // 16)
        def _(q):
            rows[j, pl.ds(q * 16, 16)] = jnp.zeros((16,), jnp.float32)

    for j in range(RPW // K):
        pltpu.sync_copy(rows, acc.at[pl.ds(s * RPW + j * K, K)])

    if with_deg:
        @pl.loop(0, N // 16)
        def _(i):
            degz[pl.ds(i * 16, 16)] = jnp.zeros((16,), jnp.float32)

        @pl.loop(0, 8)
        def _(i):
            ones[pl.ds(i * 16, 16)] = jnp.full((16,), 1.0, jnp.float32)

        @pl.when(s == 0)
        def _():
            pltpu.sync_copy(degz, dega)

    pltpu.sync_copy(src3.at[w], srcb)
    pltpu.sync_copy(dst3.at[w], dstb)
    plsc.subcore_barrier()

    @pl.loop(0, CH)
    def _(j):
        pltpu.sync_copy(feat.at[srcb.at[j]], rows)
        pltpu.sync_copy(rows, acc.at[dstb.at[j]], add=True)
        if with_deg:
            pltpu.sync_copy(ones.at[pl.ds(0, K)], dega.at[dstb.at[j]],
                            add=True)

    plsc.subcore_barrier()
    pltpu.sync_copy(acc.at[pl.ds(s * RPW, RPW)],
                    out_p.at[pl.ds(c * N + s * RPW, RPW)])
    if with_deg:
        @pl.when(s == 0)
        def _():
            pltpu.sync_copy(dega, deg_out.at[c])


def _make_seg_sum(with_deg):
    mesh = plsc.VectorSubcoreMesh(core_axis_name="c", subcore_axis_name="s")
    outs = [jax.ShapeDtypeStruct((NC * N, D), jnp.float32)]
    scratch = [
        pltpu.VMEM_SHARED((N, D), jnp.float32),
        pltpu.VMEM((CH, K), jnp.int32),
        pltpu.VMEM((CH, K), jnp.int32),
        pltpu.VMEM((K, D), jnp.float32),
    ]
    if with_deg:
        outs.append(jax.ShapeDtypeStruct((NC, N), jnp.float32))
        scratch += [
            pltpu.VMEM_SHARED((N,), jnp.float32),
            pltpu.VMEM((N,), jnp.float32),
            pltpu.VMEM((128,), jnp.float32),
        ]
    return pl.kernel(
        functools.partial(_seg_sum_body, with_deg),
        out_type=tuple(outs) if with_deg else outs[0],
        mesh=mesh,
        scratch_types=scratch,
    )


def _combine_body(relu, p0, p1, x, d0, d1, wl, wr, b, o):
    deg = jnp.maximum(d0[...] + d1[...], 1.0)
    mean = (p0[...] + p1[...]) / deg
    acc = lax.dot_general(mean, wl[...], (((1,), (1,)), ((), ())),
                          preferred_element_type=jnp.float32)
    acc = acc + lax.dot_general(x[...], wr[...], (((1,), (1,)), ((), ())),
                                preferred_element_type=jnp.float32)
    acc = acc + b[...]
    o[...] = jnp.maximum(acc, 0.0) if relu else acc


def _combine(p, x, d0, d1, wl, wr, b, relu):
    bn = 1000
    nb = N // bn
    return pl.pallas_call(
        functools.partial(_combine_body, relu),
        grid=(nb,),
        in_specs=[
            pl.BlockSpec((bn, D), lambda i: (i, 0)),
            pl.BlockSpec((bn, D), lambda i: (i + nb, 0)),
            pl.BlockSpec((bn, D), lambda i: (i, 0)),
            pl.BlockSpec((bn, 1), lambda i: (i, 0)),
            pl.BlockSpec((bn, 1), lambda i: (i, 0)),
            pl.BlockSpec((D, D), lambda i: (0, 0)),
            pl.BlockSpec((D, D), lambda i: (0, 0)),
            pl.BlockSpec((1, D), lambda i: (0, 0)),
        ],
        out_specs=pl.BlockSpec((bn, D), lambda i: (i, 0)),
        out_shape=jax.ShapeDtypeStruct((N, D), jnp.float32),
    )(p, p, x, d0, d1, wl, wr, b)


def kernel(x, edge_index, W1l, W1r, b1, W2l, W2r, b2):
    src3 = edge_index[0].astype(jnp.int32).reshape(NW, CH, K)
    dst3 = edge_index[1].astype(jnp.int32).reshape(NW, CH, K)
    b1r = b1.reshape(1, D)
    b2r = b2.reshape(1, D)

    p1, degp = _make_seg_sum(True)(x, src3, dst3)
    d0 = degp[0].reshape(N, 1)
    d1 = degp[1].reshape(N, 1)

    h = _combine(p1, x, d0, d1, W1l, W1r, b1r, relu=True)
    p2 = _make_seg_sum(False)(h, src3, dst3)
    out = _combine(p2, h, d0, d1, W2l, W2r, b2r, relu=False)
    return out


# speedup vs baseline: 8.9035x; 8.9035x over previous
import functools

import jax
import jax.numpy as jnp
from jax import lax
from jax.experimental import pallas as pl
from jax.experimental.pallas import tpu as pltpu
from jax.experimental.pallas import tpu_sc as plsc

N = 10000
E = 320000
D = 128

NC = 2
NS = 16
NW = NC * NS
EPW = E // NW
K = 125
CH = EPW // K
RPW = N // NS


def _seg_sum_body(with_deg, feat, src3, dst3, out_p0, out_p1, *rest):
    if with_deg:
        deg_out0, deg_out1, acc, srcb, dstb, rows, dega, degz, ones = rest
    else:
        acc, srcb, dstb, rows = rest
    c = lax.axis_index("c")
    s = lax.axis_index("s")
    w = c * NS + s

    @pl.loop(0, K)
    def _(j):
        @pl.loop(0, D // 16)
        def _(q):
            rows[j, pl.ds(q * 16, 16)] = jnp.zeros((16,), jnp.float32)

    for j in range(RPW // K):
        pltpu.sync_copy(rows, acc.at[pl.ds(s * RPW + j * K, K)])

    if with_deg:
        @pl.loop(0, N // 16)
        def _(i):
            degz[pl.ds(i * 16, 16)] = jnp.zeros((16,), jnp.float32)

        @pl.loop(0, 8)
        def _(i):
            ones[pl.ds(i * 16, 16)] = jnp.full((16,), 1.0, jnp.float32)

        @pl.when(s == 0)
        def _():
            pltpu.sync_copy(degz, dega)

    pltpu.sync_copy(src3.at[w], srcb)
    pltpu.sync_copy(dst3.at[w], dstb)
    plsc.subcore_barrier()

    @pl.loop(0, CH)
    def _(j):
        pltpu.sync_copy(feat.at[srcb.at[j]], rows)
        pltpu.sync_copy(rows, acc.at[dstb.at[j]], add=True)
        if with_deg:
            pltpu.sync_copy(ones.at[pl.ds(0, K)], dega.at[dstb.at[j]],
                            add=True)

    plsc.subcore_barrier()
    wb = 624
    tail = N - (NS - 1) * wb
    for cc, out_p in ((0, out_p0), (1, out_p1)):
        @pl.when(c == cc)
        def _():
            @pl.when(s < NS - 1)
            def _():
                pltpu.sync_copy(acc.at[pl.ds(s * wb, wb)],
                                out_p.at[pl.ds(s * wb, wb)])

            @pl.when(s == NS - 1)
            def _():
                pltpu.sync_copy(acc.at[pl.ds((NS - 1) * wb, tail)],
                                out_p.at[pl.ds((NS - 1) * wb, tail)])

    if with_deg:
        for cc, deg_out in ((0, deg_out0), (1, deg_out1)):
            @pl.when(jnp.logical_and(s == 0, c == cc))
            def _():
                pltpu.sync_copy(dega, deg_out)


def _make_seg_sum(with_deg):
    mesh = plsc.VectorSubcoreMesh(core_axis_name="c", subcore_axis_name="s")
    outs = [jax.ShapeDtypeStruct((N, D), jnp.float32),
            jax.ShapeDtypeStruct((N, D), jnp.float32)]
    scratch = [
        pltpu.VMEM_SHARED((N, D), jnp.float32),
        pltpu.VMEM((CH, K), jnp.int32),
        pltpu.VMEM((CH, K), jnp.int32),
        pltpu.VMEM((K, D), jnp.float32),
    ]
    if with_deg:
        outs += [jax.ShapeDtypeStruct((N,), jnp.float32),
                 jax.ShapeDtypeStruct((N,), jnp.float32)]
        scratch += [
            pltpu.VMEM_SHARED((N,), jnp.float32),
            pltpu.VMEM((N,), jnp.float32),
            pltpu.VMEM((128,), jnp.float32),
        ]
    return pl.kernel(
        functools.partial(_seg_sum_body, with_deg),
        out_type=tuple(outs),
        mesh=mesh,
        scratch_types=scratch,
    )


def _combine_body(relu, p0, p1, x, d0, d1, wl, wr, b, o):
    deg = jnp.maximum(d0[...] + d1[...], 1.0)
    mean = (p0[...] + p1[...]) / deg
    acc = lax.dot_general(mean, wl[...], (((1,), (1,)), ((), ())),
                          preferred_element_type=jnp.float32)
    acc = acc + lax.dot_general(x[...], wr[...], (((1,), (1,)), ((), ())),
                                preferred_element_type=jnp.float32)
    acc = acc + b[...]
    o[...] = jnp.maximum(acc, 0.0) if relu else acc


def _combine(p0, p1, x, d0, d1, wl, wr, b, relu):
    bn = 1000
    nb = N // bn
    return pl.pallas_call(
        functools.partial(_combine_body, relu),
        grid=(nb,),
        in_specs=[
            pl.BlockSpec((bn, D), lambda i: (i, 0)),
            pl.BlockSpec((bn, D), lambda i: (i, 0)),
            pl.BlockSpec((bn, D), lambda i: (i, 0)),
            pl.BlockSpec((bn, 1), lambda i: (i, 0)),
            pl.BlockSpec((bn, 1), lambda i: (i, 0)),
            pl.BlockSpec((D, D), lambda i: (0, 0)),
            pl.BlockSpec((D, D), lambda i: (0, 0)),
            pl.BlockSpec((1, D), lambda i: (0, 0)),
        ],
        out_specs=pl.BlockSpec((bn, D), lambda i: (i, 0)),
        out_shape=jax.ShapeDtypeStruct((N, D), jnp.float32),
    )(p0, p1, x, d0, d1, wl, wr, b)


def kernel(x, edge_index, W1l, W1r, b1, W2l, W2r, b2):
    src3 = edge_index[0].astype(jnp.int32).reshape(NW, CH, K)
    dst3 = edge_index[1].astype(jnp.int32).reshape(NW, CH, K)
    b1r = b1.reshape(1, D)
    b2r = b2.reshape(1, D)

    p10, p11, deg0, deg1 = _make_seg_sum(True)(x, src3, dst3)
    d0 = deg0.reshape(N, 1)
    d1 = deg1.reshape(N, 1)

    h = _combine(p10, p11, x, d0, d1, W1l, W1r, b1r, relu=True)
    p20, p21 = _make_seg_sum(False)(h, src3, dst3)
    out = _combine(p20, p21, h, d0, d1, W2l, W2r, b2r, relu=False)
    return out


# speedup vs baseline: 10.0257x; 1.1260x over previous
import functools

import jax
import jax.numpy as jnp
from jax import lax
from jax.experimental import pallas as pl
from jax.experimental.pallas import tpu as pltpu
from jax.experimental.pallas import tpu_sc as plsc

N = 10000
E = 320000
D = 128

NC = 2
NS = 16
NW = NC * NS
EPW = E // NW
K = 80
CH = EPW // K
RPW = N // NS


def _seg_sum_body(with_deg, feat, src3, dst3, *rest):
    if with_deg:
        (zeros1d, out_p0, out_p1, deg_out0, deg_out1, acc, srcb, dstb,
         rows0, rows1, semg0, semg1, sems0, sems1, dega, ones) = rest
    else:
        (out_p0, out_p1, acc, srcb, dstb,
         rows0, rows1, semg0, semg1, sems0, sems1) = rest
    c = lax.axis_index("c")
    s = lax.axis_index("s")
    w = c * NS + s

    @pl.loop(0, K)
    def _(j):
        @pl.loop(0, D // 16)
        def _(q):
            rows0[j, pl.ds(q * 16, 16)] = jnp.zeros((16,), jnp.float32)

    for j in range(RPW // K):
        pltpu.sync_copy(rows0, acc.at[pl.ds(s * RPW + j * K, K)])
    rem = RPW - (RPW // K) * K
    if rem:
        pltpu.sync_copy(rows0.at[pl.ds(0, rem)],
                        acc.at[pl.ds(s * RPW + RPW - rem, rem)])

    if with_deg:
        @pl.loop(0, 8)
        def _(i):
            ones[pl.ds(i * 16, 16)] = jnp.full((16,), 1.0, jnp.float32)

        @pl.when(s == 0)
        def _():
            pltpu.sync_copy(zeros1d, dega)

    pltpu.sync_copy(src3.at[pl.ds(w * EPW, EPW)], srcb)
    pltpu.sync_copy(dst3.at[w], dstb)
    plsc.subcore_barrier()

    pltpu.sync_copy(feat.at[srcb.at[pl.ds(0, K)]], rows0)
    pltpu.sync_copy(rows0, acc.at[dstb.at[0]], add=True)
    if with_deg:
        pltpu.sync_copy(ones.at[pl.ds(0, K)], dega.at[dstb.at[0]], add=True)
    pltpu.async_copy(feat.at[srcb.at[pl.ds(K, K)]], rows0, semg0)

    @pl.loop(0, (CH - 1) // 2)
    def _(i):
        a = 2 * i + 1
        b = a + 1
        pltpu.make_async_copy(feat.at[srcb.at[pl.ds(a * K, K)]], rows0, semg0).wait()

        @pl.when(i > 0)
        def _():
            pltpu.make_async_copy(rows1, acc.at[dstb.at[0]], sems1).wait()

        pltpu.async_copy(feat.at[srcb.at[pl.ds(b * K, K)]], rows1, semg1)
        pltpu.async_copy(rows0, acc.at[dstb.at[a]], sems0, add=True)
        if with_deg:
            pltpu.sync_copy(ones.at[pl.ds(0, K)], dega.at[dstb.at[a]],
                            add=True)
        pltpu.make_async_copy(feat.at[srcb.at[pl.ds(b * K, K)]], rows1, semg1).wait()
        pltpu.make_async_copy(rows0, acc.at[dstb.at[a]], sems0).wait()

        @pl.when(i < (CH - 1) // 2 - 1)
        def _():
            pltpu.async_copy(feat.at[srcb.at[pl.ds((a + 2) * K, K)]], rows0, semg0)

        pltpu.async_copy(rows1, acc.at[dstb.at[b]], sems1, add=True)
        if with_deg:
            pltpu.sync_copy(ones.at[pl.ds(0, K)], dega.at[dstb.at[b]],
                            add=True)

    pltpu.make_async_copy(rows1, acc.at[dstb.at[0]], sems1).wait()
    plsc.subcore_barrier()
    wb = 624
    tail = N - (NS - 1) * wb
    for cc, out_p in ((0, out_p0), (1, out_p1)):
        @pl.when(c == cc)
        def _():
            @pl.when(s < NS - 1)
            def _():
                pltpu.sync_copy(acc.at[pl.ds(s * wb, wb)],
                                out_p.at[pl.ds(s * wb, wb)])

            @pl.when(s == NS - 1)
            def _():
                pltpu.sync_copy(acc.at[pl.ds((NS - 1) * wb, tail)],
                                out_p.at[pl.ds((NS - 1) * wb, tail)])

    if with_deg:
        for cc, deg_out in ((0, deg_out0), (1, deg_out1)):
            @pl.when(jnp.logical_and(s == 0, c == cc))
            def _():
                pltpu.sync_copy(dega, deg_out)


def _make_seg_sum(with_deg):
    mesh = plsc.VectorSubcoreMesh(core_axis_name="c", subcore_axis_name="s")
    outs = [jax.ShapeDtypeStruct((N, D), jnp.float32),
            jax.ShapeDtypeStruct((N, D), jnp.float32)]
    scratch = [
        pltpu.VMEM_SHARED((N, D), jnp.float32),
        pltpu.VMEM((EPW,), jnp.int32),
        pltpu.VMEM((CH, K), jnp.int32),
        pltpu.VMEM((K, D), jnp.float32),
        pltpu.VMEM((K, D), jnp.float32),
        pltpu.SemaphoreType.DMA,
        pltpu.SemaphoreType.DMA,
        pltpu.SemaphoreType.DMA,
        pltpu.SemaphoreType.DMA,
    ]
    if with_deg:
        outs += [jax.ShapeDtypeStruct((N,), jnp.float32),
                 jax.ShapeDtypeStruct((N,), jnp.float32)]
        scratch += [
            pltpu.VMEM_SHARED((N,), jnp.float32),
            pltpu.VMEM((128,), jnp.float32),
        ]
    return pl.kernel(
        functools.partial(_seg_sum_body, with_deg),
        out_type=tuple(outs),
        mesh=mesh,
        scratch_types=scratch,
    )


def _combine_body(relu, p0, p1, x, d0, d1, wl, wr, b, o):
    deg = jnp.maximum(d0[...] + d1[...], 1.0)
    mean = (p0[...] + p1[...]) / deg
    acc = lax.dot_general(mean, wl[...], (((1,), (1,)), ((), ())),
                          preferred_element_type=jnp.float32)
    acc = acc + lax.dot_general(x[...], wr[...], (((1,), (1,)), ((), ())),
                                preferred_element_type=jnp.float32)
    acc = acc + b[...]
    o[...] = jnp.maximum(acc, 0.0) if relu else acc


def _combine(p0, p1, x, d0, d1, wl, wr, b, relu):
    bn = 1000
    nb = N // bn
    return pl.pallas_call(
        functools.partial(_combine_body, relu),
        grid=(nb,),
        in_specs=[
            pl.BlockSpec((bn, D), lambda i: (i, 0)),
            pl.BlockSpec((bn, D), lambda i: (i, 0)),
            pl.BlockSpec((bn, D), lambda i: (i, 0)),
            pl.BlockSpec((bn, 1), lambda i: (i, 0)),
            pl.BlockSpec((bn, 1), lambda i: (i, 0)),
            pl.BlockSpec((D, D), lambda i: (0, 0)),
            pl.BlockSpec((D, D), lambda i: (0, 0)),
            pl.BlockSpec((1, D), lambda i: (0, 0)),
        ],
        out_specs=pl.BlockSpec((bn, D), lambda i: (i, 0)),
        out_shape=jax.ShapeDtypeStruct((N, D), jnp.float32),
    )(p0, p1, x, d0, d1, wl, wr, b)


def kernel(x, edge_index, W1l, W1r, b1, W2l, W2r, b2):
    src3 = edge_index[0].astype(jnp.int32)
    dst3 = edge_index[1].astype(jnp.int32).reshape(NW, CH, K)
    b1r = b1.reshape(1, D)
    b2r = b2.reshape(1, D)

    p10, p11, deg0, deg1 = _make_seg_sum(True)(
        x, src3, dst3, jnp.zeros((N,), jnp.float32))
    d0 = deg0.reshape(N, 1)
    d1 = deg1.reshape(N, 1)

    h = _combine(p10, p11, x, d0, d1, W1l, W1r, b1r, relu=True)
    p20, p21 = _make_seg_sum(False)(h, src3, dst3)
    out = _combine(p20, p21, h, d0, d1, W2l, W2r, b2r, relu=False)
    return out


# speedup vs baseline: 11.5991x; 1.1569x over previous
import functools

import jax
import jax.numpy as jnp
from jax import lax
from jax.experimental import pallas as pl
from jax.experimental.pallas import tpu as pltpu
from jax.experimental.pallas import tpu_sc as plsc

N = 10000
E = 320000
D = 128

NC = 2
NS = 16
NW = NC * NS
EPW = E // NW
K = 128
PADE = 240
EPWP = EPW + PADE
CH = EPWP // K
NPAD = 128
NA = N + NPAD
ZPW = NA // NS
HALF = CH // 2 * K


def _seg_sum_body(with_deg, feat, srcf, dst2, *rest):
    if with_deg:
        (zeros1d, out_p0, out_p1, deg_out0, deg_out1, acc, srcb, dstb,
         rows0, rows1, semg0, semg1, sems0, sems1, dega, ones) = rest
    else:
        (out_p0, out_p1, acc, srcb, dstb,
         rows0, rows1, semg0, semg1, sems0, sems1) = rest
    c = lax.axis_index("c")
    s = lax.axis_index("s")
    w = c * NS + s

    @pl.loop(0, K)
    def _(j):
        @pl.loop(0, D // 16)
        def _(q):
            rows0[j, pl.ds(q * 16, 16)] = jnp.zeros((16,), jnp.float32)

    for j in range(ZPW // K):
        pltpu.sync_copy(rows0, acc.at[pl.ds(s * ZPW + j * K, K)])
    rem = ZPW - (ZPW // K) * K
    if rem:
        pltpu.sync_copy(rows0.at[pl.ds(0, rem)],
                        acc.at[pl.ds(s * ZPW + ZPW - rem, rem)])

    if with_deg:
        @pl.loop(0, K // 16)
        def _(i):
            ones[pl.ds(i * 16, 16)] = jnp.full((16,), 1.0, jnp.float32)

        @pl.when(s == 0)
        def _():
            pltpu.sync_copy(zeros1d, dega)

    pltpu.sync_copy(srcf.at[pl.ds(w * EPWP, HALF)], srcb)
    pltpu.sync_copy(dst2.at[pl.ds(w * CH, CH)], dstb)
    plsc.subcore_barrier()

    def gather(chunk_mod, rbuf, sem):
        pltpu.async_copy(
            feat.at[srcb.at[pl.ds(chunk_mod * K, K)]], rbuf, sem)

    def gwait(rbuf, sem):
        pltpu.make_async_copy(feat.at[srcb.at[pl.ds(0, K)]], rbuf, sem).wait()

    def scat(chunk, rbuf, sem):
        pltpu.async_copy(rbuf, acc.at[dstb.at[chunk]], sem, add=True)

    def swait(rbuf, sem):
        pltpu.make_async_copy(rbuf, acc.at[dstb.at[0]], sem).wait()

    gather(0, rows0, semg0)
    HI = CH // 2

    @pl.loop(0, HI)
    def _(i):
        a = 2 * i
        b = a + 1
        gwait(rows0, semg0)

        @pl.when(i > 0)
        def _():
            swait(rows1, sems1)

        gather(lax.rem(b, HI), rows1, semg1)
        scat(a, rows0, sems0)
        if with_deg:
            pltpu.sync_copy(ones, dega.at[dstb.at[a]], add=True)
        gwait(rows1, semg1)

        @pl.when(i == HI // 2 - 1)
        def _():
            pltpu.sync_copy(srcf.at[pl.ds(w * EPWP + HALF, HALF)], srcb)

        swait(rows0, sems0)

        @pl.when(i < HI - 1)
        def _():
            gather(lax.rem(a + 2, HI), rows0, semg0)

        scat(b, rows1, sems1)
        if with_deg:
            pltpu.sync_copy(ones, dega.at[dstb.at[b]], add=True)

    swait(rows1, sems1)
    plsc.subcore_barrier()
    wb = 624
    tail = N - (NS - 1) * wb
    for cc, out_p in ((0, out_p0), (1, out_p1)):
        @pl.when(c == cc)
        def _():
            @pl.when(s < NS - 1)
            def _():
                pltpu.sync_copy(acc.at[pl.ds(s * wb, wb)],
                                out_p.at[pl.ds(s * wb, wb)])

            @pl.when(s == NS - 1)
            def _():
                pltpu.sync_copy(acc.at[pl.ds((NS - 1) * wb, tail)],
                                out_p.at[pl.ds((NS - 1) * wb, tail)])

    if with_deg:
        for cc, deg_out in ((0, deg_out0), (1, deg_out1)):
            @pl.when(jnp.logical_and(s == 0, c == cc))
            def _():
                pltpu.sync_copy(dega, deg_out)


def _make_seg_sum(with_deg):
    mesh = plsc.VectorSubcoreMesh(core_axis_name="c", subcore_axis_name="s")
    outs = [jax.ShapeDtypeStruct((N, D), jnp.float32),
            jax.ShapeDtypeStruct((N, D), jnp.float32)]
    scratch = [
        pltpu.VMEM_SHARED((NA, D), jnp.float32),
        pltpu.VMEM((HALF,), jnp.int32),
        pltpu.VMEM((CH, K), jnp.int32),
        pltpu.VMEM((K, D), jnp.float32),
        pltpu.VMEM((K, D), jnp.float32),
        pltpu.SemaphoreType.DMA,
        pltpu.SemaphoreType.DMA,
        pltpu.SemaphoreType.DMA,
        pltpu.SemaphoreType.DMA,
    ]
    if with_deg:
        outs += [jax.ShapeDtypeStruct((NA,), jnp.float32),
                 jax.ShapeDtypeStruct((NA,), jnp.float32)]
        scratch += [
            pltpu.VMEM_SHARED((NA,), jnp.float32),
            pltpu.VMEM((K,), jnp.float32),
        ]
    return pl.kernel(
        functools.partial(_seg_sum_body, with_deg),
        out_type=tuple(outs),
        mesh=mesh,
        scratch_types=scratch,
    )


def _combine_body(relu, p0, p1, x, d0, d1, wl, wr, b, o):
    deg = jnp.maximum(d0[...] + d1[...], 1.0)
    mean = (p0[...] + p1[...]) / deg
    acc = lax.dot_general(mean, wl[...], (((1,), (1,)), ((), ())),
                          preferred_element_type=jnp.float32)
    acc = acc + lax.dot_general(x[...], wr[...], (((1,), (1,)), ((), ())),
                                preferred_element_type=jnp.float32)
    acc = acc + b[...]
    o[...] = jnp.maximum(acc, 0.0) if relu else acc


def _combine(p0, p1, x, d0, d1, wl, wr, b, relu):
    bn = 1000
    nb = N // bn
    return pl.pallas_call(
        functools.partial(_combine_body, relu),
        grid=(nb,),
        in_specs=[
            pl.BlockSpec((bn, D), lambda i: (i, 0)),
            pl.BlockSpec((bn, D), lambda i: (i, 0)),
            pl.BlockSpec((bn, D), lambda i: (i, 0)),
            pl.BlockSpec((bn, 1), lambda i: (i, 0)),
            pl.BlockSpec((bn, 1), lambda i: (i, 0)),
            pl.BlockSpec((D, D), lambda i: (0, 0)),
            pl.BlockSpec((D, D), lambda i: (0, 0)),
            pl.BlockSpec((1, D), lambda i: (0, 0)),
        ],
        out_specs=pl.BlockSpec((bn, D), lambda i: (i, 0)),
        out_shape=jax.ShapeDtypeStruct((N, D), jnp.float32),
    )(p0, p1, x, d0, d1, wl, wr, b)


def kernel(x, edge_index, W1l, W1r, b1, W2l, W2r, b2):
    src = edge_index[0].astype(jnp.int32).reshape(NW, EPW)
    dst = edge_index[1].astype(jnp.int32).reshape(NW, EPW)
    ii = jnp.arange(NW * PADE, dtype=jnp.int32).reshape(NW, PADE)
    src_pad = jnp.concatenate([src, (ii * 131) % N], axis=1).reshape(-1)
    dst_pad = jnp.concatenate([dst, N + (ii % NPAD)], axis=1)
    dst_pad = dst_pad.reshape(NW * CH, K)
    b1r = b1.reshape(1, D)
    b2r = b2.reshape(1, D)

    p10, p11, deg0, deg1 = _make_seg_sum(True)(
        x, src_pad, dst_pad, jnp.zeros((NA,), jnp.float32))
    d0 = deg0[:N].reshape(N, 1)
    d1 = deg1[:N].reshape(N, 1)

    h = _combine(p10, p11, x, d0, d1, W1l, W1r, b1r, relu=True)
    p20, p21 = _make_seg_sum(False)(h, src_pad, dst_pad)
    out = _combine(p20, p21, h, d0, d1, W2l, W2r, b2r, relu=False)
    return out


# speedup vs baseline: 11.8662x; 1.0230x over previous
import functools

import jax
import jax.numpy as jnp
from jax import lax
from jax.experimental import pallas as pl
from jax.experimental.pallas import tpu as pltpu
from jax.experimental.pallas import tpu_sc as plsc

N = 10000
E = 320000
D = 128

NC = 2
NS = 16
NW = NC * NS
EPW = E // NW
K = 128
PADE = 240
EPWP = EPW + PADE
CH = EPWP // K
NPAD = 128
NA = N + NPAD
ZPW = NA // NS
HALF = CH // 2 * K


def _seg_sum_body(with_deg, feat, srcf, dst2, *rest):
    if with_deg:
        (zeros1d, out_p0, out_p1, deg_out0, deg_out1, acc, srcb, dstb,
         rows0, rows1, semg0, semg1, sems0, sems1, dega, ones) = rest
    else:
        (out_p0, out_p1, acc, srcb, dstb,
         rows0, rows1, semg0, semg1, sems0, sems1) = rest
    c = lax.axis_index("c")
    s = lax.axis_index("s")
    w = c * NS + s

    pltpu.async_copy(srcf.at[pl.ds(w * EPWP, HALF)], srcb, semg0)
    pltpu.async_copy(dst2.at[pl.ds(w * CH, CH)], dstb, semg1)

    @pl.loop(0, K)
    def _(j):
        for q in range(D // 16):
            rows0[j, pl.ds(q * 16, 16)] = jnp.zeros((16,), jnp.float32)

    for j in range(ZPW // K):
        pltpu.sync_copy(rows0, acc.at[pl.ds(s * ZPW + j * K, K)])
    rem = ZPW - (ZPW // K) * K
    if rem:
        pltpu.sync_copy(rows0.at[pl.ds(0, rem)],
                        acc.at[pl.ds(s * ZPW + ZPW - rem, rem)])

    if with_deg:
        @pl.loop(0, K // 16)
        def _(i):
            ones[pl.ds(i * 16, 16)] = jnp.full((16,), 1.0, jnp.float32)

        @pl.when(s == 0)
        def _():
            pltpu.sync_copy(zeros1d, dega)

    pltpu.make_async_copy(srcf.at[pl.ds(w * EPWP, HALF)], srcb, semg0).wait()
    pltpu.make_async_copy(dst2.at[pl.ds(w * CH, CH)], dstb, semg1).wait()
    plsc.subcore_barrier()

    def gather(chunk_mod, rbuf, sem):
        pltpu.async_copy(
            feat.at[srcb.at[pl.ds(chunk_mod * K, K)]], rbuf, sem)

    def gwait(rbuf, sem):
        pltpu.make_async_copy(feat.at[srcb.at[pl.ds(0, K)]], rbuf, sem).wait()

    def scat(chunk, rbuf, sem):
        pltpu.async_copy(rbuf, acc.at[dstb.at[chunk]], sem, add=True)

    def swait(rbuf, sem):
        pltpu.make_async_copy(rbuf, acc.at[dstb.at[0]], sem).wait()

    gather(0, rows0, semg0)
    HI = CH // 2

    @pl.loop(0, HI)
    def _(i):
        a = 2 * i
        b = a + 1
        gwait(rows0, semg0)

        @pl.when(i > 0)
        def _():
            swait(rows1, sems1)

        gather(lax.rem(b, HI), rows1, semg1)
        scat(a, rows0, sems0)
        if with_deg:
            pltpu.sync_copy(ones, dega.at[dstb.at[a]], add=True)
        gwait(rows1, semg1)

        @pl.when(i == HI // 2 - 1)
        def _():
            pltpu.sync_copy(srcf.at[pl.ds(w * EPWP + HALF, HALF)], srcb)

        swait(rows0, sems0)

        @pl.when(i < HI - 1)
        def _():
            gather(lax.rem(a + 2, HI), rows0, semg0)

        scat(b, rows1, sems1)
        if with_deg:
            pltpu.sync_copy(ones, dega.at[dstb.at[b]], add=True)

    swait(rows1, sems1)
    plsc.subcore_barrier()
    wb = 624
    tail = N - (NS - 1) * wb
    for cc, out_p in ((0, out_p0), (1, out_p1)):
        @pl.when(c == cc)
        def _():
            @pl.when(s < NS - 1)
            def _():
                pltpu.sync_copy(acc.at[pl.ds(s * wb, wb)],
                                out_p.at[pl.ds(s * wb, wb)])

            @pl.when(s == NS - 1)
            def _():
                pltpu.sync_copy(acc.at[pl.ds((NS - 1) * wb, tail)],
                                out_p.at[pl.ds((NS - 1) * wb, tail)])

    if with_deg:
        for cc, deg_out in ((0, deg_out0), (1, deg_out1)):
            @pl.when(jnp.logical_and(s == 0, c == cc))
            def _():
                pltpu.sync_copy(dega, deg_out)


def _make_seg_sum(with_deg):
    mesh = plsc.VectorSubcoreMesh(core_axis_name="c", subcore_axis_name="s")
    outs = [jax.ShapeDtypeStruct((N, D), jnp.float32),
            jax.ShapeDtypeStruct((N, D), jnp.float32)]
    scratch = [
        pltpu.VMEM_SHARED((NA, D), jnp.float32),
        pltpu.VMEM((HALF,), jnp.int32),
        pltpu.VMEM((CH, K), jnp.int32),
        pltpu.VMEM((K, D), jnp.float32),
        pltpu.VMEM((K, D), jnp.float32),
        pltpu.SemaphoreType.DMA,
        pltpu.SemaphoreType.DMA,
        pltpu.SemaphoreType.DMA,
        pltpu.SemaphoreType.DMA,
    ]
    if with_deg:
        outs += [jax.ShapeDtypeStruct((NA,), jnp.float32),
                 jax.ShapeDtypeStruct((NA,), jnp.float32)]
        scratch += [
            pltpu.VMEM_SHARED((NA,), jnp.float32),
            pltpu.VMEM((K,), jnp.float32),
        ]
    return pl.kernel(
        functools.partial(_seg_sum_body, with_deg),
        out_type=tuple(outs),
        mesh=mesh,
        scratch_types=scratch,
    )


def _combine_body(relu, p0, p1, x, d0, d1, wl, wr, b, o):
    deg = jnp.maximum(d0[...] + d1[...], 1.0)
    mean = (p0[...] + p1[...]) / deg
    acc = lax.dot_general(mean, wl[...], (((1,), (1,)), ((), ())),
                          preferred_element_type=jnp.float32)
    acc = acc + lax.dot_general(x[...], wr[...], (((1,), (1,)), ((), ())),
                                preferred_element_type=jnp.float32)
    acc = acc + b[...]
    o[...] = jnp.maximum(acc, 0.0) if relu else acc


def _combine(p0, p1, x, d0, d1, wl, wr, b, relu):
    bn = 2000
    nb = N // bn
    return pl.pallas_call(
        functools.partial(_combine_body, relu),
        grid=(nb,),
        in_specs=[
            pl.BlockSpec((bn, D), lambda i: (i, 0)),
            pl.BlockSpec((bn, D), lambda i: (i, 0)),
            pl.BlockSpec((bn, D), lambda i: (i, 0)),
            pl.BlockSpec((bn, 1), lambda i: (i, 0)),
            pl.BlockSpec((bn, 1), lambda i: (i, 0)),
            pl.BlockSpec((D, D), lambda i: (0, 0)),
            pl.BlockSpec((D, D), lambda i: (0, 0)),
            pl.BlockSpec((1, D), lambda i: (0, 0)),
        ],
        out_specs=pl.BlockSpec((bn, D), lambda i: (i, 0)),
        out_shape=jax.ShapeDtypeStruct((N, D), jnp.float32),
    )(p0, p1, x, d0, d1, wl, wr, b)


def kernel(x, edge_index, W1l, W1r, b1, W2l, W2r, b2):
    src = edge_index[0].astype(jnp.int32).reshape(NW, EPW)
    dst = edge_index[1].astype(jnp.int32).reshape(NW, EPW)
    ii = jnp.arange(NW * PADE, dtype=jnp.int32).reshape(NW, PADE)
    src_pad = jnp.concatenate([src, (ii * 131) % N], axis=1).reshape(-1)
    dst_pad = jnp.concatenate([dst, N + (ii % NPAD)], axis=1)
    dst_pad = dst_pad.reshape(NW * CH, K)
    b1r = b1.reshape(1, D)
    b2r = b2.reshape(1, D)

    p10, p11, deg0, deg1 = _make_seg_sum(True)(
        x, src_pad, dst_pad, jnp.zeros((NA,), jnp.float32))
    d0 = deg0[:N].reshape(N, 1)
    d1 = deg1[:N].reshape(N, 1)

    h = _combine(p10, p11, x, d0, d1, W1l, W1r, b1r, relu=True)
    p20, p21 = _make_seg_sum(False)(h, src_pad, dst_pad)
    out = _combine(p20, p21, h, d0, d1, W2l, W2r, b2r, relu=False)
    return out


# speedup vs baseline: 12.2142x; 1.0293x over previous
import functools

import jax
import jax.numpy as jnp
from jax import lax
from jax.experimental import pallas as pl
from jax.experimental.pallas import tpu as pltpu
from jax.experimental.pallas import tpu_sc as plsc

N = 10000
E = 320000
D = 128

NC = 2
NS = 16
NW = NC * NS
EPW = E // NW
K = 128
PADE = 240
EPWP = EPW + PADE
CH = EPWP // K
NPAD = 128
NA = N + NPAD
ZPW = NA // NS
HALF = CH // 2 * K


def _seg_sum_body(with_deg, feat, srcf, dst2, *rest):
    if with_deg:
        (zeros1d, out_p0, out_p1, deg_out0, deg_out1, acc, srcb, dstb,
         rows0, rows1, semg0, semg1, sems0, sems1, dega, ones) = rest
    else:
        (out_p0, out_p1, acc, srcb, dstb,
         rows0, rows1, semg0, semg1, sems0, sems1) = rest
    c = lax.axis_index("c")
    s = lax.axis_index("s")
    w = c * NS + s

    pltpu.async_copy(srcf.at[pl.ds(w * EPWP, HALF)], srcb, semg0)
    pltpu.async_copy(dst2.at[pl.ds(w * CH, CH)], dstb, semg1)

    @pl.loop(0, K)
    def _(j):
        for q in range(D // 16):
            rows0[j, pl.ds(q * 16, 16)] = jnp.zeros((16,), jnp.float32)

    for j in range(ZPW // K):
        pltpu.sync_copy(rows0, acc.at[pl.ds(s * ZPW + j * K, K)])
    rem = ZPW - (ZPW // K) * K
    if rem:
        pltpu.sync_copy(rows0.at[pl.ds(0, rem)],
                        acc.at[pl.ds(s * ZPW + ZPW - rem, rem)])

    if with_deg:
        @pl.loop(0, K // 16)
        def _(i):
            ones[pl.ds(i * 16, 16)] = jnp.full((16,), 1.0, jnp.float32)

        @pl.when(s == 0)
        def _():
            pltpu.sync_copy(zeros1d, dega)

    pltpu.make_async_copy(srcf.at[pl.ds(w * EPWP, HALF)], srcb, semg0).wait()
    pltpu.make_async_copy(dst2.at[pl.ds(w * CH, CH)], dstb, semg1).wait()
    plsc.subcore_barrier()

    def gather(chunk_mod, rbuf, sem):
        pltpu.async_copy(
            feat.at[srcb.at[pl.ds(chunk_mod * K, K)]], rbuf, sem)

    def gwait(rbuf, sem):
        pltpu.make_async_copy(feat.at[srcb.at[pl.ds(0, K)]], rbuf, sem).wait()

    def scat(chunk, rbuf, sem):
        pltpu.async_copy(rbuf, acc.at[dstb.at[chunk]], sem, add=True)

    def swait(rbuf, sem):
        pltpu.make_async_copy(rbuf, acc.at[dstb.at[0]], sem).wait()

    gather(0, rows0, semg0)
    HI = CH // 2

    @pl.loop(0, HI)
    def _(i):
        a = 2 * i
        b = a + 1
        gwait(rows0, semg0)

        @pl.when(i > 0)
        def _():
            swait(rows1, sems1)

        gather(lax.rem(b, HI), rows1, semg1)
        scat(a, rows0, sems0)
        if with_deg:
            pltpu.sync_copy(ones, dega.at[dstb.at[a]], add=True)
        gwait(rows1, semg1)

        @pl.when(i == HI // 2 - 1)
        def _():
            pltpu.sync_copy(srcf.at[pl.ds(w * EPWP + HALF, HALF)], srcb)

        swait(rows0, sems0)

        @pl.when(i < HI - 1)
        def _():
            gather(lax.rem(a + 2, HI), rows0, semg0)

        scat(b, rows1, sems1)
        if with_deg:
            pltpu.sync_copy(ones, dega.at[dstb.at[b]], add=True)

    swait(rows1, sems1)
    plsc.subcore_barrier()
    wb = 624
    tail = N - (NS - 1) * wb
    for cc, out_p in ((0, out_p0), (1, out_p1)):
        @pl.when(c == cc)
        def _():
            @pl.when(s < NS - 1)
            def _():
                pltpu.sync_copy(acc.at[pl.ds(s * wb, wb)],
                                out_p.at[pl.ds(s * wb, wb)])

            @pl.when(s == NS - 1)
            def _():
                pltpu.sync_copy(acc.at[pl.ds((NS - 1) * wb, tail)],
                                out_p.at[pl.ds((NS - 1) * wb, tail)])

    if with_deg:
        for cc, deg_out in ((0, deg_out0), (1, deg_out1)):
            @pl.when(jnp.logical_and(s == 0, c == cc))
            def _():
                pltpu.sync_copy(dega, deg_out)


def _make_seg_sum(with_deg):
    mesh = plsc.VectorSubcoreMesh(core_axis_name="c", subcore_axis_name="s")
    outs = [jax.ShapeDtypeStruct((N, D), jnp.float32),
            jax.ShapeDtypeStruct((N, D), jnp.float32)]
    scratch = [
        pltpu.VMEM_SHARED((NA, D), jnp.float32),
        pltpu.VMEM((HALF,), jnp.int32),
        pltpu.VMEM((CH, K), jnp.int32),
        pltpu.VMEM((K, D), jnp.float32),
        pltpu.VMEM((K, D), jnp.float32),
        pltpu.SemaphoreType.DMA,
        pltpu.SemaphoreType.DMA,
        pltpu.SemaphoreType.DMA,
        pltpu.SemaphoreType.DMA,
    ]
    if with_deg:
        outs += [jax.ShapeDtypeStruct((NA,), jnp.float32),
                 jax.ShapeDtypeStruct((NA,), jnp.float32)]
        scratch += [
            pltpu.VMEM_SHARED((NA,), jnp.float32),
            pltpu.VMEM((K,), jnp.float32),
        ]
    return pl.kernel(
        functools.partial(_seg_sum_body, with_deg),
        out_type=tuple(outs),
        mesh=mesh,
        scratch_types=scratch,
    )


def _dinv_body(d0, d1, o):
    d = d0[...] + d1[...]
    v = 1.0 / jnp.maximum(d, 1.0)
    o[...] = v[:N].reshape(N, 1)


def _dinv(deg0, deg1):
    return pl.pallas_call(
        _dinv_body,
        in_specs=[pl.BlockSpec((NA,), lambda: (0,)),
                  pl.BlockSpec((NA,), lambda: (0,))],
        out_specs=pl.BlockSpec((N, 1), lambda: (0, 0)),
        out_shape=jax.ShapeDtypeStruct((N, 1), jnp.float32),
    )(deg0, deg1)


def _combine_body(relu, p0, p1, x, dinv, wl, wr, b, o):
    mean = (p0[...] + p1[...]) * dinv[...]
    acc = lax.dot_general(mean, wl[...], (((1,), (1,)), ((), ())),
                          preferred_element_type=jnp.float32)
    acc = acc + lax.dot_general(x[...], wr[...], (((1,), (1,)), ((), ())),
                                preferred_element_type=jnp.float32)
    acc = acc + b[...]
    o[...] = jnp.maximum(acc, 0.0) if relu else acc


def _combine(p0, p1, x, dinv, wl, wr, b, relu):
    bn = 2000
    nb = N // bn
    return pl.pallas_call(
        functools.partial(_combine_body, relu),
        grid=(nb,),
        in_specs=[
            pl.BlockSpec((bn, D), lambda i: (i, 0)),
            pl.BlockSpec((bn, D), lambda i: (i, 0)),
            pl.BlockSpec((bn, D), lambda i: (i, 0)),
            pl.BlockSpec((bn, 1), lambda i: (i, 0)),
            pl.BlockSpec((D, D), lambda i: (0, 0)),
            pl.BlockSpec((D, D), lambda i: (0, 0)),
            pl.BlockSpec((1, D), lambda i: (0, 0)),
        ],
        out_specs=pl.BlockSpec((bn, D), lambda i: (i, 0)),
        out_shape=jax.ShapeDtypeStruct((N, D), jnp.float32),
    )(p0, p1, x, dinv, wl, wr, b)


def kernel(x, edge_index, W1l, W1r, b1, W2l, W2r, b2):
    src = edge_index[0].astype(jnp.int32).reshape(NW, EPW)
    dst = edge_index[1].astype(jnp.int32).reshape(NW, EPW)
    ii = jnp.arange(NW * PADE, dtype=jnp.int32).reshape(NW, PADE)
    src_pad = jnp.concatenate([src, (ii * 131) % N], axis=1).reshape(-1)
    dst_pad = jnp.concatenate([dst, N + (ii % NPAD)], axis=1)
    dst_pad = dst_pad.reshape(NW * CH, K)
    b1r = b1.reshape(1, D)
    b2r = b2.reshape(1, D)

    p10, p11, deg0, deg1 = _make_seg_sum(True)(
        x, src_pad, dst_pad, jnp.zeros((NA,), jnp.float32))
    dinv = _dinv(deg0, deg1)

    h = _combine(p10, p11, x, dinv, W1l, W1r, b1r, relu=True)
    p20, p21 = _make_seg_sum(False)(h, src_pad, dst_pad)
    out = _combine(p20, p21, h, dinv, W2l, W2r, b2r, relu=False)
    return out


# speedup vs baseline: 12.6416x; 1.0350x over previous
import functools

import jax
import jax.numpy as jnp
from jax import lax
from jax.experimental import pallas as pl
from jax.experimental.pallas import tpu as pltpu
from jax.experimental.pallas import tpu_sc as plsc

N = 10000
E = 320000
D = 128

NC = 2
NS = 16
NW = NC * NS
EPW = E // NW
K = 128
PADE = 240
EPWP = EPW + PADE
CH = EPWP // K
NPAD = 128
NA = N + NPAD
ZPW = NA // NS
HALF = CH // 2 * K


def _seg_sum_body(with_deg, feat, ei3, *rest):
    if with_deg:
        (zeros1d, out_p0, out_p1, deg_out0, deg_out1, acc,
         ib0, ib1, ib2, ib3, rows0, rows1,
         semi0, semi1, semi2, semi3, semg0, semg1, sems0, sems1,
         dega, ones) = rest
    else:
        (out_p0, out_p1, acc, ib0, ib1, ib2, ib3, rows0, rows1,
         semi0, semi1, semi2, semi3, semg0, semg1, sems0, sems1) = rest
    c = lax.axis_index("c")
    s = lax.axis_index("s")
    w = c * NS + s
    cb = w * CH

    def idx(chunk, ib, sem):
        pltpu.async_copy(ei3.at[chunk], ib, sem)

    def iwait(ib, sem):
        pltpu.make_async_copy(ei3.at[0], ib, sem).wait()

    def gather(ib, rbuf, sem):
        pltpu.async_copy(feat.at[ib.at[0]], rbuf, sem)

    def gwait(rbuf, sem):
        pltpu.make_async_copy(feat.at[ib0.at[0]], rbuf, sem).wait()

    def scat(ib, rbuf, sem):
        pltpu.async_copy(rbuf, acc.at[ib.at[1]], sem, add=True)

    def swait(rbuf, sem):
        pltpu.make_async_copy(rbuf, acc.at[ib0.at[1]], sem).wait()

    def dscat(ib):
        pltpu.sync_copy(ones, dega.at[ib.at[1]], add=True)

    idx(cb + 0, ib0, semi0)
    idx(cb + 1, ib1, semi1)
    idx(cb + 2, ib2, semi2)
    idx(cb + 3, ib3, semi3)

    @pl.loop(0, K)
    def _(j):
        for q in range(D // 16):
            rows0[j, pl.ds(q * 16, 16)] = jnp.zeros((16,), jnp.float32)

    for j in range(ZPW // K):
        pltpu.sync_copy(rows0, acc.at[pl.ds(s * ZPW + j * K, K)])
    rem = ZPW - (ZPW // K) * K
    if rem:
        pltpu.sync_copy(rows0.at[pl.ds(0, rem)],
                        acc.at[pl.ds(s * ZPW + ZPW - rem, rem)])

    if with_deg:
        @pl.loop(0, K // 16)
        def _(i):
            ones[pl.ds(i * 16, 16)] = jnp.full((16,), 1.0, jnp.float32)

        @pl.when(s == 0)
        def _():
            pltpu.sync_copy(zeros1d, dega)

    plsc.subcore_barrier()
    iwait(ib0, semi0)
    gather(ib0, rows0, semg0)
    HI = CH // 4

    @pl.loop(0, HI)
    def _(i):
        c4 = cb + 4 * i + 4
        gwait(rows0, semg0)

        @pl.when(i > 0)
        def _():
            swait(rows1, sems1)
            idx(c4 - 1, ib3, semi3)

        iwait(ib1, semi1)
        gather(ib1, rows1, semg1)
        scat(ib0, rows0, sems0)
        if with_deg:
            dscat(ib0)
        gwait(rows1, semg1)
        swait(rows0, sems0)

        @pl.when(i < HI - 1)
        def _():
            idx(c4, ib0, semi0)

        iwait(ib2, semi2)
        gather(ib2, rows0, semg0)
        scat(ib1, rows1, sems1)
        if with_deg:
            dscat(ib1)
        gwait(rows0, semg0)
        swait(rows1, sems1)

        @pl.when(i < HI - 1)
        def _():
            idx(c4 + 1, ib1, semi1)

        iwait(ib3, semi3)
        gather(ib3, rows1, semg1)
        scat(ib2, rows0, sems0)
        if with_deg:
            dscat(ib2)
        gwait(rows1, semg1)
        swait(rows0, sems0)

        @pl.when(i < HI - 1)
        def _():
            idx(c4 + 2, ib2, semi2)
            iwait(ib0, semi0)
            gather(ib0, rows0, semg0)

        scat(ib3, rows1, sems1)
        if with_deg:
            dscat(ib3)

    swait(rows1, sems1)
    plsc.subcore_barrier()
    wb = 624
    tail = N - (NS - 1) * wb
    for cc, out_p in ((0, out_p0), (1, out_p1)):
        @pl.when(c == cc)
        def _():
            @pl.when(s < NS - 1)
            def _():
                pltpu.sync_copy(acc.at[pl.ds(s * wb, wb)],
                                out_p.at[pl.ds(s * wb, wb)])

            @pl.when(s == NS - 1)
            def _():
                pltpu.sync_copy(acc.at[pl.ds((NS - 1) * wb, tail)],
                                out_p.at[pl.ds((NS - 1) * wb, tail)])

    if with_deg:
        for cc, deg_out in ((0, deg_out0), (1, deg_out1)):
            @pl.when(jnp.logical_and(s == 0, c == cc))
            def _():
                pltpu.sync_copy(dega, deg_out)


def _make_seg_sum(with_deg):
    mesh = plsc.VectorSubcoreMesh(core_axis_name="c", subcore_axis_name="s")
    outs = [jax.ShapeDtypeStruct((N, D), jnp.float32),
            jax.ShapeDtypeStruct((N, D), jnp.float32)]
    scratch = [
        pltpu.VMEM_SHARED((NA, D), jnp.float32),
        pltpu.VMEM((2, K), jnp.int32),
        pltpu.VMEM((2, K), jnp.int32),
        pltpu.VMEM((2, K), jnp.int32),
        pltpu.VMEM((2, K), jnp.int32),
        pltpu.VMEM((K, D), jnp.float32),
        pltpu.VMEM((K, D), jnp.float32),
        pltpu.SemaphoreType.DMA,
        pltpu.SemaphoreType.DMA,
        pltpu.SemaphoreType.DMA,
        pltpu.SemaphoreType.DMA,
        pltpu.SemaphoreType.DMA,
        pltpu.SemaphoreType.DMA,
        pltpu.SemaphoreType.DMA,
        pltpu.SemaphoreType.DMA,
    ]
    if with_deg:
        outs += [jax.ShapeDtypeStruct((NA,), jnp.float32),
                 jax.ShapeDtypeStruct((NA,), jnp.float32)]
        scratch += [
            pltpu.VMEM_SHARED((NA,), jnp.float32),
            pltpu.VMEM((K,), jnp.float32),
        ]
    return pl.kernel(
        functools.partial(_seg_sum_body, with_deg),
        out_type=tuple(outs),
        mesh=mesh,
        scratch_types=scratch,
    )


def _dinv_body(d0, d1, o):
    d = d0[...] + d1[...]
    v = 1.0 / jnp.maximum(d, 1.0)
    o[...] = v[:N].reshape(N, 1)


def _dinv(deg0, deg1):
    return pl.pallas_call(
        _dinv_body,
        in_specs=[pl.BlockSpec((NA,), lambda: (0,)),
                  pl.BlockSpec((NA,), lambda: (0,))],
        out_specs=pl.BlockSpec((N, 1), lambda: (0, 0)),
        out_shape=jax.ShapeDtypeStruct((N, 1), jnp.float32),
    )(deg0, deg1)


def _combine_body(relu, p0, p1, x, dinv, wl, wr, b, o):
    mean = (p0[...] + p1[...]) * dinv[...]
    acc = lax.dot_general(mean, wl[...], (((1,), (1,)), ((), ())),
                          preferred_element_type=jnp.float32)
    acc = acc + lax.dot_general(x[...], wr[...], (((1,), (1,)), ((), ())),
                                preferred_element_type=jnp.float32)
    acc = acc + b[...]
    o[...] = jnp.maximum(acc, 0.0) if relu else acc


def _combine(p0, p1, x, dinv, wl, wr, b, relu):
    bn = 2000
    nb = N // bn
    return pl.pallas_call(
        functools.partial(_combine_body, relu),
        grid=(nb,),
        in_specs=[
            pl.BlockSpec((bn, D), lambda i: (i, 0)),
            pl.BlockSpec((bn, D), lambda i: (i, 0)),
            pl.BlockSpec((bn, D), lambda i: (i, 0)),
            pl.BlockSpec((bn, 1), lambda i: (i, 0)),
            pl.BlockSpec((D, D), lambda i: (0, 0)),
            pl.BlockSpec((D, D), lambda i: (0, 0)),
            pl.BlockSpec((1, D), lambda i: (0, 0)),
        ],
        out_specs=pl.BlockSpec((bn, D), lambda i: (i, 0)),
        out_shape=jax.ShapeDtypeStruct((N, D), jnp.float32),
    )(p0, p1, x, dinv, wl, wr, b)


def kernel(x, edge_index, W1l, W1r, b1, W2l, W2r, b2):
    ii = jnp.arange(NW * PADE, dtype=jnp.int32).reshape(1, NW * PADE)
    pad = jnp.concatenate([(ii * 131) % N, N + (ii % NPAD)], axis=0)
    ei3 = jnp.concatenate([edge_index.astype(jnp.int32), pad], axis=1)
    ei3 = ei3.reshape(2, NW * CH, K).transpose(1, 0, 2)
    b1r = b1.reshape(1, D)
    b2r = b2.reshape(1, D)

    p10, p11, deg0, deg1 = _make_seg_sum(True)(
        x, ei3, jnp.zeros((NA,), jnp.float32))
    dinv = _dinv(deg0, deg1)

    h = _combine(p10, p11, x, dinv, W1l, W1r, b1r, relu=True)
    p20, p21 = _make_seg_sum(False)(h, ei3)
    out = _combine(p20, p21, h, dinv, W2l, W2r, b2r, relu=False)
    return out
